# Initial kernel scaffold; baseline (speedup 1.0000x reference)
import jax
import jax.numpy as jnp
from jax.experimental import pallas as pl


def kernel(x, table, W, b):
    raise NotImplementedError("write your pallas kernel here")


# speedup vs baseline: 28.1497x; 28.1497x over previous
import functools

import jax
import jax.numpy as jnp
from jax import lax
from jax.experimental import pallas as pl
from jax.experimental.pallas import tpu as pltpu
from jax.experimental.pallas import tpu_sc as plsc

EMB_SIZE = 1000000
EMB_DIM = 16
HIST = 50
BATCH = 16384
FC_IN = 800
FC_OUT = 5

N_IDX = BATCH * HIST
NC, NS = 2, 16
NW = NC * NS
PER_W = N_IDX // NW
CHUNK = 3200
NCH = PER_W // CHUNK


def _gather_body(idx_hbm, table_hbm, out_hbm, idx_v, rows_v, sem):
    wid = lax.axis_index("s") * NC + lax.axis_index("c")
    base = wid * PER_W

    def step(i, carry):
        off = base + i * CHUNK
        pltpu.sync_copy(idx_hbm.at[pl.ds(off, CHUNK)], idx_v)
        pltpu.async_copy(table_hbm.at[idx_v], rows_v, sem).wait()
        pltpu.sync_copy(rows_v, out_hbm.at[pl.ds(off, CHUNK)])
        return carry

    lax.fori_loop(0, NCH, step, 0)


def _sc_gather(idx_flat, table):
    mesh = plsc.VectorSubcoreMesh(core_axis_name="c", subcore_axis_name="s")
    kern = functools.partial(
        pl.kernel,
        mesh=mesh,
        out_type=jax.ShapeDtypeStruct((N_IDX, EMB_DIM), jnp.float32),
        scratch_types=[
            pltpu.VMEM((CHUNK,), jnp.int32),
            pltpu.VMEM((CHUNK, EMB_DIM), jnp.float32),
            pltpu.SemaphoreType.DMA,
        ],
        compiler_params=pltpu.CompilerParams(use_tc_tiling_on_sc=False),
    )(_gather_body)
    return kern(idx_flat, table)


def _mm_body(emb_ref, w_ref, b_ref, out_ref):
    logits = (
        jnp.dot(emb_ref[...], w_ref[...], preferred_element_type=jnp.float32)
        + b_ref[...]
    )
    m = jnp.max(logits, axis=-1, keepdims=True)
    e = jnp.exp(logits - m)
    out_ref[...] = e / jnp.sum(e, axis=-1, keepdims=True)


def _tc_dense(emb, w_t, b2):
    bm = 1024
    grid = BATCH // bm
    return pl.pallas_call(
        _mm_body,
        grid=(grid,),
        in_specs=[
            pl.BlockSpec((bm, FC_IN), lambda i: (i, 0)),
            pl.BlockSpec((FC_IN, FC_OUT), lambda i: (0, 0)),
            pl.BlockSpec((1, FC_OUT), lambda i: (0, 0)),
        ],
        out_specs=pl.BlockSpec((bm, FC_OUT), lambda i: (i, 0)),
        out_shape=jax.ShapeDtypeStruct((BATCH, FC_OUT), jnp.float32),
    )(emb, w_t, b2)


def kernel(x, table, W, b):
    idx_flat = x.reshape(-1).astype(jnp.int32)
    emb = _sc_gather(idx_flat, table)
    emb = emb.reshape(BATCH, FC_IN)
    return _tc_dense(emb, W.T, b.reshape(1, FC_OUT))


# speedup vs baseline: 38.1359x; 1.3548x over previous
import functools

import jax
import jax.numpy as jnp
from jax import lax
from jax.experimental import pallas as pl
from jax.experimental.pallas import tpu as pltpu
from jax.experimental.pallas import tpu_sc as plsc

EMB_SIZE = 1000000
EMB_DIM = 16
HIST = 50
BATCH = 16384
FC_IN = 800
FC_OUT = 5

N_IDX = BATCH * HIST
NC, NS = 2, 16
NW = NC * NS
PER_W = N_IDX // NW
CHUNK = 3200
NCH = PER_W // CHUNK


def _gather_body(idx_hbm, table_hbm, out_hbm, idx_v, rows_v, sem):
    wid = lax.axis_index("s") * NC + lax.axis_index("c")
    base = wid * PER_W

    def step(i, carry):
        off = base + i * CHUNK
        pltpu.sync_copy(idx_hbm.at[pl.ds(off, CHUNK)], idx_v)
        pltpu.async_copy(table_hbm.at[idx_v], rows_v, sem).wait()
        pltpu.sync_copy(rows_v, out_hbm.at[pl.ds(off, CHUNK)])
        return carry

    lax.fori_loop(0, NCH, step, 0)


def _sc_gather(idx_flat, table):
    mesh = plsc.VectorSubcoreMesh(core_axis_name="c", subcore_axis_name="s")
    kern = functools.partial(
        pl.kernel,
        mesh=mesh,
        out_type=jax.ShapeDtypeStruct((N_IDX, EMB_DIM), jnp.float32),
        scratch_types=[
            pltpu.VMEM((CHUNK,), jnp.int32),
            pltpu.VMEM((CHUNK, EMB_DIM), jnp.float32),
            pltpu.SemaphoreType.DMA,
        ],
        compiler_params=pltpu.CompilerParams(use_tc_tiling_on_sc=False),
    )(_gather_body)
    return kern(idx_flat, table)


_TBK = 65536
_TW = _TBK // 8
_TGRID = 16
_VPAD = _TGRID * _TBK
_XBK = N_IDX // 128 // _TGRID


def _tr_body(tt_ref, x_ref, out_ref, xo_ref):
    for s in range(8):
        out_ref[:, s * EMB_DIM:(s + 1) * EMB_DIM] = (
            tt_ref[:, s * _TW:(s + 1) * _TW].T
        )
    v = x_ref[...]
    band = v & ~(_TBK - 1)
    rem = v - band
    xo_ref[...] = band + ((rem & (_TW - 1)) << 3) + (rem >> 13)


def _tc_relayout(table_t, x2):
    flat, xr = pl.pallas_call(
        _tr_body,
        grid=(_TGRID,),
        in_specs=[
            pl.BlockSpec((EMB_DIM, _TBK), lambda i: (0, i)),
            pl.BlockSpec((_XBK, 128), lambda i: (i, 0)),
        ],
        out_specs=[
            pl.BlockSpec((_TW, 128), lambda i: (i, 0)),
            pl.BlockSpec((_XBK, 128), lambda i: (i, 0)),
        ],
        out_shape=[
            jax.ShapeDtypeStruct((_VPAD * EMB_DIM // 128, 128), jnp.float32),
            jax.ShapeDtypeStruct((N_IDX // 128, 128), jnp.int32),
        ],
        compiler_params=pltpu.CompilerParams(vmem_limit_bytes=100 * 1024 * 1024),
    )(table_t, x2)
    return flat.reshape(_VPAD, EMB_DIM), xr.reshape(N_IDX)


def _mm_body(emb_ref, w_ref, b_ref, out_ref):
    logits = (
        jnp.dot(emb_ref[...], w_ref[...], preferred_element_type=jnp.float32)
        + b_ref[...]
    )
    m = jnp.max(logits, axis=-1, keepdims=True)
    e = jnp.exp(logits - m)
    out_ref[...] = e / jnp.sum(e, axis=-1, keepdims=True)


def _tc_dense(emb, w_t, b2):
    bm = 1024
    grid = BATCH // bm
    return pl.pallas_call(
        _mm_body,
        grid=(grid,),
        in_specs=[
            pl.BlockSpec((bm, FC_IN), lambda i: (i, 0)),
            pl.BlockSpec((FC_IN, FC_OUT), lambda i: (0, 0)),
            pl.BlockSpec((1, FC_OUT), lambda i: (0, 0)),
        ],
        out_specs=pl.BlockSpec((bm, FC_OUT), lambda i: (i, 0)),
        out_shape=jax.ShapeDtypeStruct((BATCH, FC_OUT), jnp.float32),
    )(emb, w_t, b2)


def kernel(x, table, W, b):
    x2 = x.reshape(-1).astype(jnp.int32).reshape(N_IDX // 128, 128)
    table_rm, idx_flat = _tc_relayout(table.T, x2)
    emb = _sc_gather(idx_flat, table_rm)
    emb = emb.reshape(BATCH, FC_IN)
    return _tc_dense(emb, W.T, b.reshape(1, FC_OUT))


# speedup vs baseline: 84.2076x; 2.2081x over previous
import functools

import jax
import jax.numpy as jnp
import numpy as np
from jax import lax
from jax.experimental import pallas as pl
from jax.experimental.pallas import tpu as pltpu
from jax.experimental.pallas import tpu_sc as plsc

EMB_SIZE = 1000000
EMB_DIM = 16
HIST = 50
BATCH = 16384
FC_IN = 800
FC_OUT = 5

N_IDX = BATCH * HIST
NC, NS = 2, 16
NW = NC * NS
PER_W = N_IDX // NW
CH = 1600
NBLK = PER_W // CH

FPAD = 1024
NSLOT = BATCH * FPAD // EMB_DIM


def _dst_slots():
    i = np.arange(N_IDX, dtype=np.int64)
    b, h = i // HIST, i % HIST
    slot = (((b // 8) * 8 + h // 8) * 8 + b % 8) * 8 + h % 8
    return slot.astype(np.int32)


_DST = _dst_slots()


def _gs_body(idx_hbm, dst_hbm, tbl_hbm, emb_hbm,
             idxv0, idxv1, dstv0, dstv1, rows0, rows1,
             gsem0, gsem1, wsem0, wsem1):
    wid = lax.axis_index("s") * NC + lax.axis_index("c")
    base = wid * PER_W
    idxv = (idxv0, idxv1)
    dstv = (dstv0, dstv1)
    rows = (rows0, rows1)
    gsems = (gsem0, gsem1)
    wsems = (wsem0, wsem1)

    def start_blk(g, d):
        off = base + g * CH
        pltpu.sync_copy(idx_hbm.at[pl.ds(off, CH)], idxv[d])
        pltpu.sync_copy(dst_hbm.at[pl.ds(off, CH)], dstv[d])
        pltpu.async_copy(tbl_hbm.at[idxv[d]], rows[d], gsems[d])

    start_blk(0, 0)

    def pair(p, carry):
        for d in range(2):
            g = 2 * p + d
            o = 1 - d

            @pl.when(g >= 1)
            def _():
                pltpu.make_async_copy(rows[o], emb_hbm.at[dstv[o]], wsems[o]).wait()

            @pl.when(g < NBLK - 1)
            def _():
                start_blk(g + 1, o)

            pltpu.make_async_copy(tbl_hbm.at[idxv[d]], rows[d], gsems[d]).wait()
            pltpu.async_copy(rows[d], emb_hbm.at[dstv[d]], wsems[d])
        return carry

    lax.fori_loop(0, NBLK // 2, pair, 0)
    pltpu.make_async_copy(rows[1], emb_hbm.at[dstv[1]], wsems[1]).wait()


def _sc_gather_scatter(idx_flat, dst, table):
    mesh = plsc.VectorSubcoreMesh(core_axis_name="c", subcore_axis_name="s")
    kern = functools.partial(
        pl.kernel,
        mesh=mesh,
        out_type=jax.ShapeDtypeStruct((NSLOT, EMB_DIM), jnp.float32),
        scratch_types=[
            pltpu.VMEM((CH,), jnp.int32),
            pltpu.VMEM((CH,), jnp.int32),
            pltpu.VMEM((CH,), jnp.int32),
            pltpu.VMEM((CH,), jnp.int32),
            pltpu.VMEM((CH, EMB_DIM), jnp.float32),
            pltpu.VMEM((CH, EMB_DIM), jnp.float32),
            pltpu.SemaphoreType.DMA,
            pltpu.SemaphoreType.DMA,
            pltpu.SemaphoreType.DMA,
            pltpu.SemaphoreType.DMA,
        ],
        compiler_params=pltpu.CompilerParams(
            use_tc_tiling_on_sc=False, needs_layout_passes=False),
    )(_gs_body)
    return kern(idx_flat, dst, table)


_BI = 1024
_BR = _BI * FPAD // 128
_NTC = 7


def _dense_body(e_ref, w_ref, b_ref, o_ref):
    e4 = e_ref[...].reshape(_BI // 8, 8, 8, 128)
    lanes = lax.broadcasted_iota(jnp.int32, (_BI, 128), 1)
    logits = jnp.broadcast_to(b_ref[...], (_BI, 128))
    for tc in range(_NTC):
        a = e4[:, tc].reshape(_BI, 128)
        if tc == _NTC - 1:
            a = jnp.where(lanes < 32, a, 0.0)
        logits = logits + jnp.dot(a, w_ref[tc],
                                  preferred_element_type=jnp.float32)
    m = jnp.max(logits, axis=1, keepdims=True)
    ex = jnp.exp(logits - m)
    o = ex / jnp.sum(ex, axis=1, keepdims=True)
    o_ref[...] = o[:, :FC_OUT]


def _tc_dense(emb_rows, w_pad, b_pad):
    return pl.pallas_call(
        _dense_body,
        grid=(BATCH // _BI,),
        in_specs=[
            pl.BlockSpec((_BR, 128), lambda i: (i, 0)),
            pl.BlockSpec((_NTC, 128, 128), lambda i: (0, 0, 0)),
            pl.BlockSpec((1, 128), lambda i: (0, 0)),
        ],
        out_specs=pl.BlockSpec((_BI, FC_OUT), lambda i: (i, 0)),
        out_shape=jax.ShapeDtypeStruct((BATCH, FC_OUT), jnp.float32),
        compiler_params=pltpu.CompilerParams(vmem_limit_bytes=100 * 1024 * 1024),
    )(emb_rows, w_pad, b_pad)


def _w_maps():
    tc = np.arange(_NTC)[:, None]
    l = np.arange(128)[None, :]
    h = tc * 8 + l // EMB_DIM
    p = h * EMB_DIM + l % EMB_DIM
    valid = h < HIST
    return np.where(valid, p, 0).astype(np.int32), \
        valid.astype(np.float32)[:, :, None]


_WPERM, _WMASK = _w_maps()

_TBK = 65536
_TW = _TBK // 8
_TGRID = 16
_VPAD = _TGRID * _TBK
_XBK = N_IDX // 128 // _TGRID


def _tr_body(tt_ref, x_ref, out_ref, xo_ref):
    stacked = jnp.concatenate(
        [tt_ref[:, s * _TW:(s + 1) * _TW] for s in range(8)], axis=0
    )
    out_ref[...] = stacked.T
    v = x_ref[...]
    band = v & ~(_TBK - 1)
    rem = v - band
    xo_ref[...] = band + ((rem & (_TW - 1)) << 3) + (rem >> 13)


def _tc_relayout(table_t, x2):
    flat, xr = pl.pallas_call(
        _tr_body,
        grid=(_TGRID,),
        in_specs=[
            pl.BlockSpec((EMB_DIM, _TBK), lambda i: (0, i)),
            pl.BlockSpec((_XBK, 128), lambda i: (i, 0)),
        ],
        out_specs=[
            pl.BlockSpec((_TW, 128), lambda i: (i, 0)),
            pl.BlockSpec((_XBK, 128), lambda i: (i, 0)),
        ],
        out_shape=[
            jax.ShapeDtypeStruct((_VPAD * EMB_DIM // 128, 128), jnp.float32),
            jax.ShapeDtypeStruct((N_IDX // 128, 128), jnp.int32),
        ],
        compiler_params=pltpu.CompilerParams(vmem_limit_bytes=100 * 1024 * 1024),
    )(table_t, x2)
    return flat.reshape(_VPAD, EMB_DIM), xr.reshape(N_IDX)


def kernel(x, table, W, b):
    x2 = x.reshape(-1).astype(jnp.int32).reshape(N_IDX // 128, 128)
    table_rm, idx_flat = _tc_relayout(table.T, x2)
    emb = _sc_gather_scatter(idx_flat, _DST, table_rm)
    emb_rows = emb.reshape(BATCH * FPAD // 128, 128)
    w_pad = jnp.pad((W.T[_WPERM] * _WMASK),
                    ((0, 0), (0, 0), (0, 128 - FC_OUT)))
    b_pad = jnp.concatenate(
        [b, jnp.full((128 - FC_OUT,), -1e30, jnp.float32)]).reshape(1, 128)
    return _tc_dense(emb_rows, w_pad, b_pad)


# speedup vs baseline: 96.3801x; 1.1446x over previous
import functools

import jax
import jax.numpy as jnp
import numpy as np
from jax import lax
from jax.experimental import pallas as pl
from jax.experimental.pallas import tpu as pltpu
from jax.experimental.pallas import tpu_sc as plsc

EMB_SIZE = 1000000
EMB_DIM = 16
HIST = 50
BATCH = 16384
FC_IN = 800
FC_OUT = 5

N_IDX = BATCH * HIST
NC, NS = 2, 16
NW = NC * NS
PER_W = N_IDX // NW
CH = 1600
NBLK = PER_W // CH

FPAD = 896
NSLOT = BATCH * FPAD // EMB_DIM


def _gs_body(idx_hbm, dst_hbm, tbl_hbm, emb_hbm,
             idxv0, idxv1, dstv0, dstv1, rows0, rows1,
             gsem0, gsem1, wsem0, wsem1):
    wid = lax.axis_index("s") * NC + lax.axis_index("c")
    base = wid * PER_W
    idxv = (idxv0, idxv1)
    dstv = (dstv0, dstv1)
    rows = (rows0, rows1)
    gsems = (gsem0, gsem1)
    wsems = (wsem0, wsem1)

    def start_blk(g, d):
        off = base + g * CH
        pltpu.sync_copy(idx_hbm.at[pl.ds(off, CH)], idxv[d])
        pltpu.sync_copy(dst_hbm.at[pl.ds(off, CH)], dstv[d])
        pltpu.async_copy(tbl_hbm.at[idxv[d]], rows[d], gsems[d])

    start_blk(0, 0)

    def pair(p, carry):
        for d in range(2):
            g = 2 * p + d
            o = 1 - d

            @pl.when(g >= 1)
            def _():
                pltpu.make_async_copy(rows[o], emb_hbm.at[dstv[o]], wsems[o]).wait()

            @pl.when(g < NBLK - 1)
            def _():
                start_blk(g + 1, o)

            pltpu.make_async_copy(tbl_hbm.at[idxv[d]], rows[d], gsems[d]).wait()
            pltpu.async_copy(rows[d], emb_hbm.at[dstv[d]], wsems[d])
        return carry

    lax.fori_loop(0, NBLK // 2, pair, 0)
    pltpu.make_async_copy(rows[1], emb_hbm.at[dstv[1]], wsems[1]).wait()


def _sc_gather_scatter(idx_flat, dst, table):
    mesh = plsc.VectorSubcoreMesh(core_axis_name="c", subcore_axis_name="s")
    kern = functools.partial(
        pl.kernel,
        mesh=mesh,
        out_type=jax.ShapeDtypeStruct((NSLOT, EMB_DIM), jnp.float32),
        scratch_types=[
            pltpu.VMEM((CH,), jnp.int32),
            pltpu.VMEM((CH,), jnp.int32),
            pltpu.VMEM((CH,), jnp.int32),
            pltpu.VMEM((CH,), jnp.int32),
            pltpu.VMEM((CH, EMB_DIM), jnp.float32),
            pltpu.VMEM((CH, EMB_DIM), jnp.float32),
            pltpu.SemaphoreType.DMA,
            pltpu.SemaphoreType.DMA,
            pltpu.SemaphoreType.DMA,
            pltpu.SemaphoreType.DMA,
        ],
        compiler_params=pltpu.CompilerParams(
            use_tc_tiling_on_sc=False, needs_layout_passes=False),
    )(_gs_body)
    return kern(idx_flat, dst, table)


_BI = 1024
_BR = _BI * FPAD // 128
_NTC = 7


def _dense_body(e_ref, w_ref, b_ref, o_ref):
    e4 = e_ref[...].reshape(_BI // 8, _NTC, 8, 128)
    lanes = lax.broadcasted_iota(jnp.int32, (_BI, 128), 1)
    logits = jnp.broadcast_to(b_ref[...], (_BI, 128))
    for tc in range(_NTC):
        a = e4[:, tc].reshape(_BI, 128)
        if tc == _NTC - 1:
            a = jnp.where(lanes < 32, a, 0.0)
        logits = logits + jnp.dot(a, w_ref[tc],
                                  preferred_element_type=jnp.float32)
    m = jnp.max(logits, axis=1, keepdims=True)
    ex = jnp.exp(logits - m)
    o = ex / jnp.sum(ex, axis=1, keepdims=True)
    o_ref[...] = o[:, :FC_OUT]


def _tc_dense(emb_rows, w_pad, b_pad):
    return pl.pallas_call(
        _dense_body,
        grid=(BATCH // _BI,),
        in_specs=[
            pl.BlockSpec((_BR, 128), lambda i: (i, 0)),
            pl.BlockSpec((_NTC, 128, 128), lambda i: (0, 0, 0)),
            pl.BlockSpec((1, 128), lambda i: (0, 0)),
        ],
        out_specs=pl.BlockSpec((_BI, FC_OUT), lambda i: (i, 0)),
        out_shape=jax.ShapeDtypeStruct((BATCH, FC_OUT), jnp.float32),
        compiler_params=pltpu.CompilerParams(vmem_limit_bytes=100 * 1024 * 1024),
    )(emb_rows, w_pad, b_pad)


def _w_maps():
    tc = np.arange(_NTC)[:, None]
    l = np.arange(128)[None, :]
    h = tc * 8 + l // EMB_DIM
    p = h * EMB_DIM + l % EMB_DIM
    valid = h < HIST
    return np.where(valid, p, 0).astype(np.int32), \
        valid.astype(np.float32)[:, :, None]


_WPERM, _WMASK = _w_maps()

_TBK = 65536
_TW = _TBK // 8
_TGRID = 16
_VPAD = _TGRID * _TBK
_XBK = N_IDX // 128 // _TGRID


def _tr_body(tt_ref, x_ref, out_ref, xo_ref, dst_ref):
    stacked = jnp.concatenate(
        [tt_ref[:, s * _TW:(s + 1) * _TW] for s in range(8)], axis=0
    )
    out_ref[...] = stacked.T
    v = x_ref[...]
    band = v & ~(_TBK - 1)
    rem = v - band
    xo_ref[...] = band + ((rem & (_TW - 1)) << 3) + (rem >> 13)
    i = (pl.program_id(0) * _XBK * 128
         + lax.broadcasted_iota(jnp.int32, (_XBK, 128), 0) * 128
         + lax.broadcasted_iota(jnp.int32, (_XBK, 128), 1))
    bq = jnp.floor((i.astype(jnp.float32) + 0.5) * (1.0 / HIST)).astype(jnp.int32)
    h = i - HIST * bq
    dst_ref[...] = ((((bq >> 3) * _NTC + (h >> 3)) << 6)
                    + ((bq & 7) << 3) + (h & 7))


def _tc_relayout(table_t, x2):
    flat, xr, dst = pl.pallas_call(
        _tr_body,
        grid=(_TGRID,),
        in_specs=[
            pl.BlockSpec((EMB_DIM, _TBK), lambda i: (0, i)),
            pl.BlockSpec((_XBK, 128), lambda i: (i, 0)),
        ],
        out_specs=[
            pl.BlockSpec((_TW, 128), lambda i: (i, 0)),
            pl.BlockSpec((_XBK, 128), lambda i: (i, 0)),
            pl.BlockSpec((_XBK, 128), lambda i: (i, 0)),
        ],
        out_shape=[
            jax.ShapeDtypeStruct((_VPAD * EMB_DIM // 128, 128), jnp.float32),
            jax.ShapeDtypeStruct((N_IDX // 128, 128), jnp.int32),
            jax.ShapeDtypeStruct((N_IDX // 128, 128), jnp.int32),
        ],
        compiler_params=pltpu.CompilerParams(vmem_limit_bytes=100 * 1024 * 1024),
    )(table_t, x2)
    return (flat.reshape(_VPAD, EMB_DIM), xr.reshape(N_IDX),
            dst.reshape(N_IDX))


def kernel(x, table, W, b):
    x2 = x.reshape(-1).astype(jnp.int32).reshape(N_IDX // 128, 128)
    table_rm, idx_flat, dst = _tc_relayout(table.T, x2)
    emb = _sc_gather_scatter(idx_flat, dst, table_rm)
    emb_rows = emb.reshape(BATCH * FPAD // 128, 128)
    w_pad = jnp.pad((W.T[_WPERM] * _WMASK),
                    ((0, 0), (0, 0), (0, 128 - FC_OUT)))
    b_pad = jnp.concatenate(
        [b, jnp.full((128 - FC_OUT,), -1e30, jnp.float32)]).reshape(1, 128)
    return _tc_dense(emb_rows, w_pad, b_pad)


# speedup vs baseline: 98.5358x; 1.0224x over previous
import functools

import jax
import jax.numpy as jnp
import numpy as np
from jax import lax
from jax.experimental import pallas as pl
from jax.experimental.pallas import tpu as pltpu
from jax.experimental.pallas import tpu_sc as plsc

EMB_SIZE = 1000000
EMB_DIM = 16
HIST = 50
BATCH = 16384
FC_IN = 800
FC_OUT = 5

N_IDX = BATCH * HIST
NC, NS = 2, 16
NW = NC * NS
PER_W = N_IDX // NW
CH = 1600
NBLK = PER_W // CH

FPAD = 896
NSLOT = BATCH * FPAD // EMB_DIM


def _gs_body(idx_hbm, dst_hbm, tbl_hbm, emb_hbm,
             idxv0, idxv1, dstv0, dstv1, rows0, rows1,
             gsem0, gsem1, wsem0, wsem1):
    wid = lax.axis_index("s") * NC + lax.axis_index("c")
    base = wid * PER_W
    idxv = (idxv0, idxv1)
    dstv = (dstv0, dstv1)
    rows = (rows0, rows1)
    gsems = (gsem0, gsem1)
    wsems = (wsem0, wsem1)

    def start_blk(g, d):
        off = base + g * CH
        pltpu.sync_copy(idx_hbm.at[pl.ds(off, CH)], idxv[d])
        pltpu.sync_copy(dst_hbm.at[pl.ds(off, CH)], dstv[d])
        pltpu.async_copy(tbl_hbm.at[idxv[d]], rows[d], gsems[d])

    start_blk(0, 0)

    def pair(p, carry):
        for d in range(2):
            g = 2 * p + d
            o = 1 - d

            @pl.when(g >= 1)
            def _():
                pltpu.make_async_copy(rows[o], emb_hbm.at[dstv[o]], wsems[o]).wait()

            @pl.when(g < NBLK - 1)
            def _():
                start_blk(g + 1, o)

            pltpu.make_async_copy(tbl_hbm.at[idxv[d]], rows[d], gsems[d]).wait()
            pltpu.async_copy(rows[d], emb_hbm.at[dstv[d]], wsems[d])
        return carry

    lax.fori_loop(0, NBLK // 2, pair, 0)
    pltpu.make_async_copy(rows[1], emb_hbm.at[dstv[1]], wsems[1]).wait()


def _sc_gather_scatter(idx_flat, dst, table):
    mesh = plsc.VectorSubcoreMesh(core_axis_name="c", subcore_axis_name="s")
    kern = functools.partial(
        pl.kernel,
        mesh=mesh,
        out_type=jax.ShapeDtypeStruct((NSLOT, EMB_DIM), jnp.float32),
        scratch_types=[
            pltpu.VMEM((CH,), jnp.int32),
            pltpu.VMEM((CH,), jnp.int32),
            pltpu.VMEM((CH,), jnp.int32),
            pltpu.VMEM((CH,), jnp.int32),
            pltpu.VMEM((CH, EMB_DIM), jnp.float32),
            pltpu.VMEM((CH, EMB_DIM), jnp.float32),
            pltpu.SemaphoreType.DMA,
            pltpu.SemaphoreType.DMA,
            pltpu.SemaphoreType.DMA,
            pltpu.SemaphoreType.DMA,
        ],
        compiler_params=pltpu.CompilerParams(
            use_tc_tiling_on_sc=False, needs_layout_passes=False),
    )(_gs_body)
    return kern(idx_flat, dst, table)


_BI = 2048
_BR = _BI * FPAD // 128
_NTC = 7


def _dense_body(e_ref, w_ref, b_ref, o_ref):
    e4 = e_ref[...].reshape(_BI // 8, _NTC, 8, 128)
    lanes = lax.broadcasted_iota(jnp.int32, (_BI, 128), 1)
    logits = jnp.broadcast_to(b_ref[...], (_BI, 128))
    for tc in range(_NTC):
        a = e4[:, tc].reshape(_BI, 128)
        if tc == _NTC - 1:
            a = jnp.where(lanes < 32, a, 0.0)
        logits = logits + jnp.dot(a, w_ref[tc],
                                  preferred_element_type=jnp.float32)
    m = jnp.max(logits, axis=1, keepdims=True)
    ex = jnp.exp(logits - m)
    o = ex / jnp.sum(ex, axis=1, keepdims=True)
    o_ref[...] = o[:, :FC_OUT]


def _tc_dense(emb_rows, w_pad, b_pad):
    return pl.pallas_call(
        _dense_body,
        grid=(BATCH // _BI,),
        in_specs=[
            pl.BlockSpec((_BR, 128), lambda i: (i, 0)),
            pl.BlockSpec((_NTC, 128, 128), lambda i: (0, 0, 0)),
            pl.BlockSpec((1, 128), lambda i: (0, 0)),
        ],
        out_specs=pl.BlockSpec((_BI, FC_OUT), lambda i: (i, 0)),
        out_shape=jax.ShapeDtypeStruct((BATCH, FC_OUT), jnp.float32),
        compiler_params=pltpu.CompilerParams(vmem_limit_bytes=100 * 1024 * 1024),
    )(emb_rows, w_pad, b_pad)


def _w_maps():
    tc = np.arange(_NTC)[:, None]
    l = np.arange(128)[None, :]
    h = tc * 8 + l // EMB_DIM
    p = h * EMB_DIM + l % EMB_DIM
    valid = h < HIST
    return np.where(valid, p, 0).astype(np.int32), \
        valid.astype(np.float32)[:, :, None]


_WPERM, _WMASK = _w_maps()

_TBK = 65536
_TW = _TBK // 8
_TGRID = 16
_VPAD = _TGRID * _TBK
_XBK = N_IDX // 128 // _TGRID


def _tr_body(tt_ref, x_ref, out_ref, xo_ref, dst_ref):
    stacked = jnp.concatenate(
        [tt_ref[:, s * _TW:(s + 1) * _TW] for s in range(8)], axis=0
    )
    out_ref[...] = stacked.T
    v = x_ref[...]
    band = v & ~(_TBK - 1)
    rem = v - band
    xo_ref[...] = band + ((rem & (_TW - 1)) << 3) + (rem >> 13)
    i = (pl.program_id(0) * _XBK * 128
         + lax.broadcasted_iota(jnp.int32, (_XBK, 128), 0) * 128
         + lax.broadcasted_iota(jnp.int32, (_XBK, 128), 1))
    bq = jnp.floor((i.astype(jnp.float32) + 0.5) * (1.0 / HIST)).astype(jnp.int32)
    h = i - HIST * bq
    dst_ref[...] = ((((bq >> 3) * _NTC + (h >> 3)) << 6)
                    + ((bq & 7) << 3) + (h & 7))


def _tc_relayout(table_t, x2):
    flat, xr, dst = pl.pallas_call(
        _tr_body,
        grid=(_TGRID,),
        in_specs=[
            pl.BlockSpec((EMB_DIM, _TBK), lambda i: (0, i)),
            pl.BlockSpec((_XBK, 128), lambda i: (i, 0)),
        ],
        out_specs=[
            pl.BlockSpec((_TW, 128), lambda i: (i, 0)),
            pl.BlockSpec((_XBK, 128), lambda i: (i, 0)),
            pl.BlockSpec((_XBK, 128), lambda i: (i, 0)),
        ],
        out_shape=[
            jax.ShapeDtypeStruct((_VPAD * EMB_DIM // 128, 128), jnp.float32),
            jax.ShapeDtypeStruct((N_IDX // 128, 128), jnp.int32),
            jax.ShapeDtypeStruct((N_IDX // 128, 128), jnp.int32),
        ],
        compiler_params=pltpu.CompilerParams(vmem_limit_bytes=100 * 1024 * 1024),
    )(table_t, x2)
    return (flat.reshape(_VPAD, EMB_DIM), xr.reshape(N_IDX),
            dst.reshape(N_IDX))


def kernel(x, table, W, b):
    x2 = x.reshape(-1).astype(jnp.int32).reshape(N_IDX // 128, 128)
    table_rm, idx_flat, dst = _tc_relayout(table.T, x2)
    emb = _sc_gather_scatter(idx_flat, dst, table_rm)
    emb_rows = emb.reshape(BATCH * FPAD // 128, 128)
    w_pad = jnp.pad((W.T[_WPERM] * _WMASK),
                    ((0, 0), (0, 0), (0, 128 - FC_OUT)))
    b_pad = jnp.concatenate(
        [b, jnp.full((128 - FC_OUT,), -1e30, jnp.float32)]).reshape(1, 128)
    return _tc_dense(emb_rows, w_pad, b_pad)


# speedup vs baseline: 98.7515x; 1.0022x over previous
import functools

import jax
import jax.numpy as jnp
import numpy as np
from jax import lax
from jax.experimental import pallas as pl
from jax.experimental.pallas import tpu as pltpu
from jax.experimental.pallas import tpu_sc as plsc

EMB_SIZE = 1000000
EMB_DIM = 16
HIST = 50
BATCH = 16384
FC_IN = 800
FC_OUT = 5

N_IDX = BATCH * HIST
NC, NS = 2, 16
NW = NC * NS
PER_W = N_IDX // NW
CH = 1600
NBLK = PER_W // CH

FPAD = 896
NSLOT = BATCH * FPAD // EMB_DIM


def _gs_body(idx_hbm, dst_hbm, tbl_hbm, emb_hbm,
             idxv0, idxv1, dstv0, dstv1, rows0, rows1,
             gsem0, gsem1, wsem0, wsem1):
    wid = lax.axis_index("s") * NC + lax.axis_index("c")
    base = wid * PER_W
    idxv = (idxv0, idxv1)
    dstv = (dstv0, dstv1)
    rows = (rows0, rows1)
    gsems = (gsem0, gsem1)
    wsems = (wsem0, wsem1)

    def start_blk(g, d):
        off = base + g * CH
        pltpu.sync_copy(idx_hbm.at[pl.ds(off, CH)], idxv[d])
        pltpu.sync_copy(dst_hbm.at[pl.ds(off, CH)], dstv[d])
        pltpu.async_copy(tbl_hbm.at[idxv[d]], rows[d], gsems[d])

    start_blk(0, 0)

    def pair(p, carry):
        for d in range(2):
            g = 2 * p + d
            o = 1 - d

            @pl.when(g >= 1)
            def _():
                pltpu.make_async_copy(rows[o], emb_hbm.at[dstv[o]], wsems[o]).wait()

            @pl.when(g < NBLK - 1)
            def _():
                start_blk(g + 1, o)

            pltpu.make_async_copy(tbl_hbm.at[idxv[d]], rows[d], gsems[d]).wait()
            pltpu.async_copy(rows[d], emb_hbm.at[dstv[d]], wsems[d])
        return carry

    lax.fori_loop(0, NBLK // 2, pair, 0)
    pltpu.make_async_copy(rows[1], emb_hbm.at[dstv[1]], wsems[1]).wait()


def _sc_gather_scatter(idx_flat, dst, table):
    mesh = plsc.VectorSubcoreMesh(core_axis_name="c", subcore_axis_name="s")
    kern = functools.partial(
        pl.kernel,
        mesh=mesh,
        out_type=jax.ShapeDtypeStruct((NSLOT, EMB_DIM), jnp.float32),
        scratch_types=[
            pltpu.VMEM((CH,), jnp.int32),
            pltpu.VMEM((CH,), jnp.int32),
            pltpu.VMEM((CH,), jnp.int32),
            pltpu.VMEM((CH,), jnp.int32),
            pltpu.VMEM((CH, EMB_DIM), jnp.float32),
            pltpu.VMEM((CH, EMB_DIM), jnp.float32),
            pltpu.SemaphoreType.DMA,
            pltpu.SemaphoreType.DMA,
            pltpu.SemaphoreType.DMA,
            pltpu.SemaphoreType.DMA,
        ],
        compiler_params=pltpu.CompilerParams(
            use_tc_tiling_on_sc=False, needs_layout_passes=False),
    )(_gs_body)
    return kern(idx_flat, dst, table)


_BI = 4096
_BR = _BI * FPAD // 128
_NTC = 7


def _dense_body(e_ref, w_ref, b_ref, o_ref):
    e4 = e_ref[...].reshape(_BI // 8, _NTC, 8, 128)
    lanes = lax.broadcasted_iota(jnp.int32, (_BI, 128), 1)
    logits = jnp.broadcast_to(b_ref[...], (_BI, 128))
    for tc in range(_NTC):
        a = e4[:, tc].reshape(_BI, 128)
        if tc == _NTC - 1:
            a = jnp.where(lanes < 32, a, 0.0)
        logits = logits + jnp.dot(a, w_ref[tc],
                                  preferred_element_type=jnp.float32)
    m = jnp.max(logits, axis=1, keepdims=True)
    ex = jnp.exp(logits - m)
    o = ex / jnp.sum(ex, axis=1, keepdims=True)
    o_ref[...] = o[:, :FC_OUT]


def _tc_dense(emb_rows, w_pad, b_pad):
    return pl.pallas_call(
        _dense_body,
        grid=(BATCH // _BI,),
        in_specs=[
            pl.BlockSpec((_BR, 128), lambda i: (i, 0)),
            pl.BlockSpec((_NTC, 128, 128), lambda i: (0, 0, 0)),
            pl.BlockSpec((1, 128), lambda i: (0, 0)),
        ],
        out_specs=pl.BlockSpec((_BI, FC_OUT), lambda i: (i, 0)),
        out_shape=jax.ShapeDtypeStruct((BATCH, FC_OUT), jnp.float32),
        compiler_params=pltpu.CompilerParams(vmem_limit_bytes=100 * 1024 * 1024),
    )(emb_rows, w_pad, b_pad)


def _w_maps():
    tc = np.arange(_NTC)[:, None]
    l = np.arange(128)[None, :]
    h = tc * 8 + l // EMB_DIM
    p = h * EMB_DIM + l % EMB_DIM
    valid = h < HIST
    return np.where(valid, p, 0).astype(np.int32), \
        valid.astype(np.float32)[:, :, None]


_WPERM, _WMASK = _w_maps()

_TBK = 65536
_TW = _TBK // 8
_TGRID = 16
_VPAD = _TGRID * _TBK
_XBK = N_IDX // 128 // _TGRID


def _tr_body(tt_ref, x_ref, out_ref, xo_ref, dst_ref):
    stacked = jnp.concatenate(
        [tt_ref[:, s * _TW:(s + 1) * _TW] for s in range(8)], axis=0
    )
    out_ref[...] = stacked.T
    v = x_ref[...]
    band = v & ~(_TBK - 1)
    rem = v - band
    xo_ref[...] = band + ((rem & (_TW - 1)) << 3) + (rem >> 13)
    i = (pl.program_id(0) * _XBK * 128
         + lax.broadcasted_iota(jnp.int32, (_XBK, 128), 0) * 128
         + lax.broadcasted_iota(jnp.int32, (_XBK, 128), 1))
    bq = jnp.floor((i.astype(jnp.float32) + 0.5) * (1.0 / HIST)).astype(jnp.int32)
    h = i - HIST * bq
    dst_ref[...] = ((((bq >> 3) * _NTC + (h >> 3)) << 6)
                    + ((bq & 7) << 3) + (h & 7))


def _tc_relayout(table_t, x2):
    flat, xr, dst = pl.pallas_call(
        _tr_body,
        grid=(_TGRID,),
        in_specs=[
            pl.BlockSpec((EMB_DIM, _TBK), lambda i: (0, i)),
            pl.BlockSpec((_XBK, 128), lambda i: (i, 0)),
        ],
        out_specs=[
            pl.BlockSpec((_TW, 128), lambda i: (i, 0)),
            pl.BlockSpec((_XBK, 128), lambda i: (i, 0)),
            pl.BlockSpec((_XBK, 128), lambda i: (i, 0)),
        ],
        out_shape=[
            jax.ShapeDtypeStruct((_VPAD * EMB_DIM // 128, 128), jnp.float32),
            jax.ShapeDtypeStruct((N_IDX // 128, 128), jnp.int32),
            jax.ShapeDtypeStruct((N_IDX // 128, 128), jnp.int32),
        ],
        compiler_params=pltpu.CompilerParams(vmem_limit_bytes=100 * 1024 * 1024),
    )(table_t, x2)
    return (flat.reshape(_VPAD, EMB_DIM), xr.reshape(N_IDX),
            dst.reshape(N_IDX))


def kernel(x, table, W, b):
    x2 = x.reshape(-1).astype(jnp.int32).reshape(N_IDX // 128, 128)
    table_rm, idx_flat, dst = _tc_relayout(table.T, x2)
    emb = _sc_gather_scatter(idx_flat, dst, table_rm)
    emb_rows = emb.reshape(BATCH * FPAD // 128, 128)
    w_pad = jnp.pad((W.T[_WPERM] * _WMASK),
                    ((0, 0), (0, 0), (0, 128 - FC_OUT)))
    b_pad = jnp.concatenate(
        [b, jnp.full((128 - FC_OUT,), -1e30, jnp.float32)]).reshape(1, 128)
    return _tc_dense(emb_rows, w_pad, b_pad)
